# baseline (speedup 1.0000x reference)
import functools

import jax
import jax.numpy as jnp
from jax import lax
from jax.experimental import pallas as pl
from jax.experimental.pallas import tpu as pltpu
from jax.experimental.pallas import tpu_sc as plsc

_HID = 256
_FD = 4
_IN = _FD + _HID


def _sc_mesh_info():
    info = plsc.get_sparse_core_info()
    return info.num_cores, info.num_subcores


def _sc_gather_rows(table, idx, chunk):
    (n,) = idx.shape
    _, d = table.shape
    dt = table.dtype
    nc, ns = _sc_mesh_info()
    nw = nc * ns
    npw = n // nw
    nchunks = npw // chunk
    mesh = plsc.VectorSubcoreMesh(core_axis_name="c", subcore_axis_name="s")

    assert nchunks % 4 == 0

    @functools.partial(
        pl.kernel,
        mesh=mesh,
        out_type=jax.ShapeDtypeStruct((n, d), dt),
        scratch_types=[
            pltpu.VMEM((npw,), jnp.int32),
            pltpu.VMEM((chunk, d), dt),
            pltpu.VMEM((chunk, d), dt),
            pltpu.VMEM((chunk, d), dt),
            pltpu.VMEM((chunk, d), dt),
            pltpu.SemaphoreType.DMA,
            pltpu.SemaphoreType.DMA,
            pltpu.SemaphoreType.DMA,
            pltpu.SemaphoreType.DMA,
        ],
    )
    def k(table_hbm, idx_hbm, out_hbm, idx_v,
          rows0, rows1, rows2, rows3, g0, g1, g2, g3):
        wid = lax.axis_index("s") * nc + lax.axis_index("c")
        base = wid * npw
        pltpu.sync_copy(idx_hbm.at[pl.ds(base, npw)], idx_v)
        rows = (rows0, rows1, rows2, rows3)
        gs = (g0, g1, g2, g3)

        def start(c, b):
            pltpu.async_copy(
                table_hbm.at[idx_v.at[pl.ds(c * chunk, chunk)]], rows[b], gs[b])

        start(0, 0)
        start(1, 1)

        def quad(i, carry):
            qq = i * 4
            for b in range(4):
                c = qq + b

                @pl.when(c + 2 < nchunks)
                def _():
                    start(c + 2, (b + 2) % 4)

                pltpu.make_async_copy(
                    table_hbm.at[idx_v.at[pl.ds(c * chunk, chunk)]],
                    rows[b], gs[b]).wait()
                pltpu.sync_copy(rows[b], out_hbm.at[pl.ds(base + c * chunk, chunk)])
            return carry

        lax.fori_loop(0, nchunks // 4, quad, 0)

    return k(table, idx)


def _sc_gather_sum(table, idx, kper, cout):
    n = idx.shape[0] // kper
    _, d = table.shape
    nc, ns = _sc_mesh_info()
    nw = nc * ns
    npw = n // nw
    chunk = cout * kper
    nchunks = npw // cout
    mesh = plsc.VectorSubcoreMesh(core_axis_name="c", subcore_axis_name="s")

    assert nchunks % 4 == 0

    @functools.partial(
        pl.kernel,
        mesh=mesh,
        out_type=jax.ShapeDtypeStruct((n, d), jnp.float32),
        scratch_types=[
            pltpu.VMEM((npw * kper,), jnp.int32),
            pltpu.VMEM((chunk, d), jnp.float32),
            pltpu.VMEM((chunk, d), jnp.float32),
            pltpu.VMEM((chunk, d), jnp.float32),
            pltpu.VMEM((chunk, d), jnp.float32),
            pltpu.VMEM((cout, d), jnp.float32),
            pltpu.SemaphoreType.DMA,
            pltpu.SemaphoreType.DMA,
            pltpu.SemaphoreType.DMA,
            pltpu.SemaphoreType.DMA,
        ],
    )
    def k(table_hbm, idx_hbm, out_hbm, idx_v,
          rows0, rows1, rows2, rows3, acc_v, g0, g1, g2, g3):
        wid = lax.axis_index("s") * nc + lax.axis_index("c")
        base = wid * npw
        pltpu.sync_copy(idx_hbm.at[pl.ds(base * kper, npw * kper)], idx_v)
        rows = (rows0, rows1, rows2, rows3)
        gs = (g0, g1, g2, g3)

        def start(c, b):
            pltpu.async_copy(
                table_hbm.at[idx_v.at[pl.ds(c * chunk, chunk)]], rows[b], gs[b])

        start(0, 0)
        start(1, 1)

        def quad(i, carry):
            qq = i * 4
            for b in range(4):
                c = qq + b

                @pl.when(c + 2 < nchunks)
                def _():
                    start(c + 2, (b + 2) % 4)

                pltpu.make_async_copy(
                    table_hbm.at[idx_v.at[pl.ds(c * chunk, chunk)]],
                    rows[b], gs[b]).wait()

                def row(r, c2):
                    for lc in range(d // 16):
                        sl = pl.ds(lc * 16, 16)
                        a = rows[b][r * kper, sl]
                        for j in range(1, kper):
                            a = a + rows[b][r * kper + j, sl]
                        acc_v[r, sl] = a
                    return c2

                lax.fori_loop(0, cout, row, 0)
                pltpu.sync_copy(acc_v, out_hbm.at[pl.ds(base + c * cout, cout)])
            return carry

        lax.fori_loop(0, nchunks // 4, quad, 0)

    return k(table, idx)


def _tc_pmat(label, f_na, wl_t, we_t):
    n = f_na.shape[0]
    dout = we_t.shape[1]
    blk = 512
    ngrid = n // blk + 1

    def body(lab_ref, fna_ref, wl_ref, we_ref, o_ref, p_ref):
        i = pl.program_id(0)

        @pl.when(i < ngrid - 1)
        def _():
            p = (
                jnp.dot(lab_ref[...], wl_ref[...], preferred_element_type=jnp.float32)
                + jnp.dot(fna_ref[...], we_ref[...], preferred_element_type=jnp.float32)
            )
            o_ref[...] = p[:, :3 * _HID]
            p_ref[...] = p[:, 3 * _HID:]

        @pl.when(i == ngrid - 1)
        def _():
            o_ref[...] = jnp.zeros((blk, 3 * _HID), jnp.float32)
            p_ref[...] = jnp.zeros((blk, _HID), jnp.float32)

    return pl.pallas_call(
        body,
        grid=(ngrid,),
        in_specs=[
            pl.BlockSpec((blk, _FD), lambda i: (jnp.minimum(i, 15), 0)),
            pl.BlockSpec((blk, _HID), lambda i: (jnp.minimum(i, 15), 0)),
            pl.BlockSpec((_FD, dout), lambda i: (0, 0)),
            pl.BlockSpec((_HID, dout), lambda i: (0, 0)),
        ],
        out_specs=[
            pl.BlockSpec((blk, 3 * _HID), lambda i: (i, 0)),
            pl.BlockSpec((blk, _HID), lambda i: (i, 0)),
        ],
        out_shape=[
            jax.ShapeDtypeStruct((ngrid * blk, 3 * _HID), jnp.float32),
            jax.ShapeDtypeStruct((ngrid * blk, _HID), jnp.float32),
        ],
    )(label, f_na, wl_t, we_t)


def _tc_gru_first(fmsg, bz, bh):
    n = fmsg.shape[0]
    blk = 1024

    def body(f_ref, bz_ref, bh_ref, o_ref):
        i = pl.program_id(0)
        z = jax.nn.sigmoid(f_ref[:, 0:_HID] + bz_ref[...])
        p = jnp.tanh(f_ref[:, 2 * _HID:3 * _HID] + bh_ref[...])
        out = z * p
        rid = lax.broadcasted_iota(jnp.int32, (blk, 1), 0) + i * blk
        o_ref[...] = jnp.where(rid == 0, 0.0, out)

    return pl.pallas_call(
        body,
        grid=(n // blk,),
        in_specs=[
            pl.BlockSpec((blk, 3 * _HID), lambda i: (i, 0)),
            pl.BlockSpec((1, _HID), lambda i: (0, 0)),
            pl.BlockSpec((1, _HID), lambda i: (0, 0)),
        ],
        out_specs=pl.BlockSpec((blk, _HID), lambda i: (i, 0)),
        out_shape=jax.ShapeDtypeStruct((n, _HID), jnp.float32),
    )(fmsg, bz, bh)


def _tc_gru_step(nei3, fmsg, wzs_t, ur_t, whs_t, bz, br, bh):
    n = fmsg.shape[0]
    blk = 1024

    def body(nei_ref, f_ref, wzs_ref, ur_ref, whs_ref, bz_ref, br_ref, bh_ref,
             o_ref):
        i = pl.program_id(0)
        nei = nei_ref[...]
        r2 = jnp.dot(nei, ur_ref[...], preferred_element_type=jnp.float32)
        n0 = nei[0:blk]
        n1 = nei[blk:2 * blk]
        n2 = nei[2 * blk:3 * blk]
        n3 = nei[3 * blk:4 * blk]
        sum_msg = n0 + n1 + n2 + n3
        rb = f_ref[:, _HID:2 * _HID] + br_ref[...]
        sg = jax.nn.sigmoid(rb + r2[0:blk]) * n0
        sg = sg + jax.nn.sigmoid(rb + r2[blk:2 * blk]) * n1
        sg = sg + jax.nn.sigmoid(rb + r2[2 * blk:3 * blk]) * n2
        sg = sg + jax.nn.sigmoid(rb + r2[3 * blk:4 * blk]) * n3
        z = jax.nn.sigmoid(
            f_ref[:, 0:_HID]
            + jnp.dot(sum_msg, wzs_ref[...], preferred_element_type=jnp.float32)
            + bz_ref[...]
        )
        pre = jnp.tanh(
            f_ref[:, 2 * _HID:3 * _HID]
            + jnp.dot(sg, whs_ref[...], preferred_element_type=jnp.float32)
            + bh_ref[...]
        )
        out = (1.0 - z) * sum_msg + z * pre
        rid = lax.broadcasted_iota(jnp.int32, (blk, 1), 0) + i * blk
        o_ref[...] = jnp.where(rid == 0, 0.0, out)

    return pl.pallas_call(
        body,
        grid=(n // blk,),
        in_specs=[
            pl.BlockSpec((4 * blk, _HID), lambda i: (i, 0)),
            pl.BlockSpec((blk, 3 * _HID), lambda i: (i, 0)),
            pl.BlockSpec((_HID, _HID), lambda i: (0, 0)),
            pl.BlockSpec((_HID, _HID), lambda i: (0, 0)),
            pl.BlockSpec((_HID, _HID), lambda i: (0, 0)),
            pl.BlockSpec((1, _HID), lambda i: (0, 0)),
            pl.BlockSpec((1, _HID), lambda i: (0, 0)),
            pl.BlockSpec((1, _HID), lambda i: (0, 0)),
        ],
        out_specs=pl.BlockSpec((blk, _HID), lambda i: (i, 0)),
        out_shape=jax.ShapeDtypeStruct((n, _HID), jnp.float32),
    )(nei3, fmsg, wzs_t, ur_t, whs_t, bz, br, bh)


def _tc_gru_last(nei3, fmsg, pout, wzs_t, ur_t, whs_t, or_t, bz, br, bh):
    n = fmsg.shape[0]
    blk = 1024

    def body(nei_ref, f_ref, p_ref, wzs_ref, ur_ref, whs_ref, or_ref,
             bz_ref, br_ref, bh_ref, o_ref, q_ref):
        i = pl.program_id(0)
        nei = nei_ref[...]
        r2 = jnp.dot(nei, ur_ref[...], preferred_element_type=jnp.float32)
        n0 = nei[0:blk]
        n1 = nei[blk:2 * blk]
        n2 = nei[2 * blk:3 * blk]
        n3 = nei[3 * blk:4 * blk]
        sum_msg = n0 + n1 + n2 + n3
        rb = f_ref[:, _HID:2 * _HID] + br_ref[...]
        sg = jax.nn.sigmoid(rb + r2[0:blk]) * n0
        sg = sg + jax.nn.sigmoid(rb + r2[blk:2 * blk]) * n1
        sg = sg + jax.nn.sigmoid(rb + r2[2 * blk:3 * blk]) * n2
        sg = sg + jax.nn.sigmoid(rb + r2[3 * blk:4 * blk]) * n3
        z = jax.nn.sigmoid(
            f_ref[:, 0:_HID]
            + jnp.dot(sum_msg, wzs_ref[...], preferred_element_type=jnp.float32)
            + bz_ref[...]
        )
        pre = jnp.tanh(
            f_ref[:, 2 * _HID:3 * _HID]
            + jnp.dot(sg, whs_ref[...], preferred_element_type=jnp.float32)
            + bh_ref[...]
        )
        out = (1.0 - z) * sum_msg + z * pre
        rid = lax.broadcasted_iota(jnp.int32, (blk, 1), 0) + i * blk
        out = jnp.where(rid == 0, 0.0, out)
        o_ref[...] = out
        q_ref[...] = p_ref[...] + jnp.dot(
            out, or_ref[...], preferred_element_type=jnp.float32)

    return pl.pallas_call(
        body,
        grid=(n // blk,),
        in_specs=[
            pl.BlockSpec((4 * blk, _HID), lambda i: (i, 0)),
            pl.BlockSpec((blk, 3 * _HID), lambda i: (i, 0)),
            pl.BlockSpec((blk, _HID), lambda i: (i, 0)),
            pl.BlockSpec((_HID, _HID), lambda i: (0, 0)),
            pl.BlockSpec((_HID, _HID), lambda i: (0, 0)),
            pl.BlockSpec((_HID, _HID), lambda i: (0, 0)),
            pl.BlockSpec((_HID, _HID), lambda i: (0, 0)),
            pl.BlockSpec((1, _HID), lambda i: (0, 0)),
            pl.BlockSpec((1, _HID), lambda i: (0, 0)),
            pl.BlockSpec((1, _HID), lambda i: (0, 0)),
        ],
        out_specs=[
            pl.BlockSpec((blk, _HID), lambda i: (i, 0)),
            pl.BlockSpec((blk, _HID), lambda i: (i, 0)),
        ],
        out_shape=[
            jax.ShapeDtypeStruct((n, _HID), jnp.float32),
            jax.ShapeDtypeStruct((n, _HID), jnp.float32),
        ],
    )(nei3, fmsg, pout, wzs_t, ur_t, whs_t, or_t, bz, br, bh)


def _tc_lstm(agg_tm, out_b, wf_t, wb_t, uf_t, ub_t, bf, bb):
    batch, seq, hh = 8, 512, 128

    def body(agg_ref, ob_ref, wf_ref, wb_ref, uf_ref, ub_ref,
             bf_ref, bb_ref, o_ref, xf_ref, xb_ref):
        nrow = batch * seq
        blk = 512
        for c in range(nrow // blk):
            h = jnp.maximum(agg_ref[pl.ds(c * blk, blk), :] + ob_ref[...], 0.0)
            xf_ref[pl.ds(c * blk, blk), :] = jnp.dot(
                h, wf_ref[...], preferred_element_type=jnp.float32) + bf_ref[...]
            xb_ref[pl.ds(c * blk, blk), :] = jnp.dot(
                h, wb_ref[...], preferred_element_type=jnp.float32) + bb_ref[...]

        uf_hi = uf_ref[...]
        ub_hi = ub_ref[...]

        def cell(g, c):
            i_ = jax.nn.sigmoid(g[:, 0:hh])
            f_ = jax.nn.sigmoid(g[:, hh:2 * hh])
            g_ = jnp.tanh(g[:, 2 * hh:3 * hh])
            o_ = jax.nn.sigmoid(g[:, 3 * hh:4 * hh])
            c = f_ * c + i_ * g_
            return o_ * jnp.tanh(c), c

        unroll = 16

        def step(i, carry):
            hf, cf, hb, cb = carry
            t0 = i * unroll
            xfb = xf_ref[pl.ds(t0 * batch, unroll * batch), :]
            xbb = xb_ref[pl.ds((seq - unroll - t0) * batch, unroll * batch), :]
            for k in range(unroll):
                kb = unroll - 1 - k
                gf = xfb[k * batch:(k + 1) * batch, :] + jnp.dot(
                    hf.astype(jnp.bfloat16), uf_hi,
                    preferred_element_type=jnp.float32)
                gb = xbb[kb * batch:(kb + 1) * batch, :] + jnp.dot(
                    hb.astype(jnp.bfloat16), ub_hi,
                    preferred_element_type=jnp.float32)
                hf, cf = cell(gf, cf)
                hb, cb = cell(gb, cb)
            return (hf, cf, hb, cb)

        z = jnp.zeros((batch, hh), jnp.float32)
        hf, cf, hb, cb = lax.fori_loop(0, seq // unroll, step, (z, z, z, z))
        o_ref[...] = jnp.concatenate([hf, hb], axis=1)

    return pl.pallas_call(
        body,
        in_specs=[
            pl.BlockSpec((batch * seq, _HID), lambda: (0, 0)),
            pl.BlockSpec((1, _HID), lambda: (0, 0)),
            pl.BlockSpec((_HID, 4 * hh), lambda: (0, 0)),
            pl.BlockSpec((_HID, 4 * hh), lambda: (0, 0)),
            pl.BlockSpec((hh, 4 * hh), lambda: (0, 0)),
            pl.BlockSpec((hh, 4 * hh), lambda: (0, 0)),
            pl.BlockSpec((1, 4 * hh), lambda: (0, 0)),
            pl.BlockSpec((1, 4 * hh), lambda: (0, 0)),
        ],
        out_specs=pl.BlockSpec((batch, 2 * hh), lambda: (0, 0)),
        out_shape=jax.ShapeDtypeStruct((batch, 2 * hh), jnp.float32),
        scratch_shapes=[
            pltpu.VMEM((seq * batch, 4 * hh), jnp.float32),
            pltpu.VMEM((seq * batch, 4 * hh), jnp.float32),
        ],
    )(agg_tm, out_b, wf_t, wb_t,
      uf_t.astype(jnp.bfloat16), ub_t.astype(jnp.bfloat16), bf, bb)


def kernel(nuc_emebedding, f_node_label, f_node_assignment, f_message,
           node_graph, message_graph, scope, diameter,
           W_z_w, W_z_b, W_r_w, U_r_w, U_r_b, W_h_w, W_h_b, out_w, out_b,
           lstm_Wih_f, lstm_Whh_f, lstm_bih_f, lstm_bhh_f,
           lstm_Wih_b, lstm_Whh_b, lstm_bih_b, lstm_bhh_b):
    hid = _HID

    nuc_pad = jnp.concatenate(
        [nuc_emebedding, jnp.zeros((1, hid), jnp.float32)], axis=0)
    fna = _sc_gather_sum(
        nuc_pad, f_node_assignment.astype(jnp.int32).reshape(-1), 8, 8)

    wcat = jnp.concatenate(
        [W_z_w[:, :_IN], W_r_w, W_h_w[:, :_IN], out_w[:, :_IN]], axis=0)
    wl_t = jnp.transpose(wcat[:, :_FD])
    we_t = jnp.transpose(wcat[:, _FD:])
    p768, pout = _tc_pmat(f_node_label, fna, wl_t, we_t)

    fmsg = _sc_gather_rows(p768, f_message.astype(jnp.int32), 32)

    bz = W_z_b.reshape(1, hid)
    br = U_r_b.reshape(1, hid)
    bh = W_h_b.reshape(1, hid)
    wzs_t = jnp.transpose(W_z_w[:, _IN:])
    ur_t = jnp.transpose(U_r_w)
    whs_t = jnp.transpose(W_h_w[:, _IN:])

    msgs = _tc_gru_first(fmsg, bz, bh)
    mg_flat = jnp.transpose(
        message_graph.astype(jnp.int32).reshape(8, 1024, 4), (0, 2, 1)).reshape(-1)
    for _ in range(3):
        nei = _sc_gather_rows(msgs, mg_flat, 64)
        msgs = _tc_gru_step(nei, fmsg, wzs_t, ur_t, whs_t, bz, br, bh)

    or_t = jnp.transpose(out_w[:, _IN:])
    nei = _sc_gather_rows(msgs, mg_flat, 64)
    msgs, q = _tc_gru_last(nei, fmsg, pout,
                           wzs_t, ur_t, whs_t, or_t, bz, br, bh)

    ng_tm = jnp.transpose(
        node_graph.astype(jnp.int32).reshape(8, 512, 4), (1, 0, 2)).reshape(-1)
    agg = _sc_gather_sum(q, ng_tm, 4, 16)

    tree = _tc_lstm(
        agg, out_b.reshape(1, hid),
        jnp.transpose(lstm_Wih_f), jnp.transpose(lstm_Wih_b),
        jnp.transpose(lstm_Whh_f), jnp.transpose(lstm_Whh_b),
        (lstm_bih_f + lstm_bhh_f).reshape(1, -1),
        (lstm_bih_b + lstm_bhh_b).reshape(1, -1))

    return (msgs, tree)


# speedup vs baseline: 1.0112x; 1.0112x over previous
import functools

import jax
import jax.numpy as jnp
from jax import lax
from jax.experimental import pallas as pl
from jax.experimental.pallas import tpu as pltpu
from jax.experimental.pallas import tpu_sc as plsc

_HID = 256
_FD = 4
_IN = _FD + _HID


def _sc_mesh_info():
    info = plsc.get_sparse_core_info()
    return info.num_cores, info.num_subcores


def _sc_gather_rows(table, idx, chunk):
    (n,) = idx.shape
    _, d = table.shape
    dt = table.dtype
    nc, ns = _sc_mesh_info()
    nw = nc * ns
    npw = n // nw
    nchunks = npw // chunk
    mesh = plsc.VectorSubcoreMesh(core_axis_name="c", subcore_axis_name="s")

    assert nchunks % 4 == 0

    @functools.partial(
        pl.kernel,
        mesh=mesh,
        out_type=jax.ShapeDtypeStruct((n, d), dt),
        scratch_types=[
            pltpu.VMEM((npw,), jnp.int32),
            pltpu.VMEM((chunk, d), dt),
            pltpu.VMEM((chunk, d), dt),
            pltpu.VMEM((chunk, d), dt),
            pltpu.VMEM((chunk, d), dt),
            pltpu.SemaphoreType.DMA,
            pltpu.SemaphoreType.DMA,
            pltpu.SemaphoreType.DMA,
            pltpu.SemaphoreType.DMA,
        ],
    )
    def k(table_hbm, idx_hbm, out_hbm, idx_v,
          rows0, rows1, rows2, rows3, g0, g1, g2, g3):
        wid = lax.axis_index("s") * nc + lax.axis_index("c")
        base = wid * npw
        pltpu.sync_copy(idx_hbm.at[pl.ds(base, npw)], idx_v)
        rows = (rows0, rows1, rows2, rows3)
        gs = (g0, g1, g2, g3)

        def start(c, b):
            pltpu.async_copy(
                table_hbm.at[idx_v.at[pl.ds(c * chunk, chunk)]], rows[b], gs[b])

        start(0, 0)
        start(1, 1)

        def quad(i, carry):
            qq = i * 4
            for b in range(4):
                c = qq + b

                @pl.when(c + 2 < nchunks)
                def _():
                    start(c + 2, (b + 2) % 4)

                pltpu.make_async_copy(
                    table_hbm.at[idx_v.at[pl.ds(c * chunk, chunk)]],
                    rows[b], gs[b]).wait()
                pltpu.sync_copy(rows[b], out_hbm.at[pl.ds(base + c * chunk, chunk)])
            return carry

        lax.fori_loop(0, nchunks // 4, quad, 0)

    return k(table, idx)


def _sc_gather_sum(table, idx, kper, cout):
    n = idx.shape[0] // kper
    _, d = table.shape
    nc, ns = _sc_mesh_info()
    nw = nc * ns
    npw = n // nw
    chunk = cout * kper
    nchunks = npw // cout
    mesh = plsc.VectorSubcoreMesh(core_axis_name="c", subcore_axis_name="s")

    @functools.partial(
        pl.kernel,
        mesh=mesh,
        out_type=jax.ShapeDtypeStruct((n, d), jnp.float32),
        scratch_types=[
            pltpu.VMEM((npw * kper,), jnp.int32),
            pltpu.VMEM((chunk, d), jnp.float32),
            pltpu.VMEM((chunk, d), jnp.float32),
            pltpu.VMEM((cout, d), jnp.float32),
            pltpu.SemaphoreType.DMA,
            pltpu.SemaphoreType.DMA,
        ],
    )
    def k(table_hbm, idx_hbm, out_hbm, idx_v, rows0, rows1, acc_v, g0, g1):
        wid = lax.axis_index("s") * nc + lax.axis_index("c")
        base = wid * npw
        pltpu.sync_copy(idx_hbm.at[pl.ds(base * kper, npw * kper)], idx_v)
        rows = (rows0, rows1)
        gs = (g0, g1)

        def start(c, b):
            pltpu.async_copy(
                table_hbm.at[idx_v.at[pl.ds(c * chunk, chunk)]], rows[b], gs[b])

        start(0, 0)

        def pair(i, carry):
            cc = i * 2
            for b in range(2):
                c = cc + b

                @pl.when(c + 1 < nchunks)
                def _():
                    start(c + 1, 1 - b)

                pltpu.make_async_copy(
                    table_hbm.at[idx_v.at[pl.ds(c * chunk, chunk)]],
                    rows[b], gs[b]).wait()

                def row(r, c2):
                    for lc in range(d // 16):
                        sl = pl.ds(lc * 16, 16)
                        a = rows[b][r * kper, sl]
                        for j in range(1, kper):
                            a = a + rows[b][r * kper + j, sl]
                        acc_v[r, sl] = a
                    return c2

                lax.fori_loop(0, cout, row, 0)
                pltpu.sync_copy(acc_v, out_hbm.at[pl.ds(base + c * cout, cout)])
            return carry

        lax.fori_loop(0, nchunks // 2, pair, 0)

    return k(table, idx)


def _tc_pmat(label, f_na, wl_t, we_t):
    n = f_na.shape[0]
    dout = we_t.shape[1]
    blk = 512
    ngrid = n // blk + 1

    def body(lab_ref, fna_ref, wl_ref, we_ref, o_ref, p_ref):
        i = pl.program_id(0)

        @pl.when(i < ngrid - 1)
        def _():
            p = (
                jnp.dot(lab_ref[...], wl_ref[...], preferred_element_type=jnp.float32)
                + jnp.dot(fna_ref[...], we_ref[...], preferred_element_type=jnp.float32)
            )
            o_ref[...] = p[:, :3 * _HID]
            p_ref[...] = p[:, 3 * _HID:]

        @pl.when(i == ngrid - 1)
        def _():
            o_ref[...] = jnp.zeros((blk, 3 * _HID), jnp.float32)
            p_ref[...] = jnp.zeros((blk, _HID), jnp.float32)

    return pl.pallas_call(
        body,
        grid=(ngrid,),
        in_specs=[
            pl.BlockSpec((blk, _FD), lambda i: (jnp.minimum(i, 15), 0)),
            pl.BlockSpec((blk, _HID), lambda i: (jnp.minimum(i, 15), 0)),
            pl.BlockSpec((_FD, dout), lambda i: (0, 0)),
            pl.BlockSpec((_HID, dout), lambda i: (0, 0)),
        ],
        out_specs=[
            pl.BlockSpec((blk, 3 * _HID), lambda i: (i, 0)),
            pl.BlockSpec((blk, _HID), lambda i: (i, 0)),
        ],
        out_shape=[
            jax.ShapeDtypeStruct((ngrid * blk, 3 * _HID), jnp.float32),
            jax.ShapeDtypeStruct((ngrid * blk, _HID), jnp.float32),
        ],
    )(label, f_na, wl_t, we_t)


def _tc_gru_first(fmsg, bz, bh):
    n = fmsg.shape[0]
    blk = 1024

    def body(f_ref, bz_ref, bh_ref, o_ref):
        i = pl.program_id(0)
        z = jax.nn.sigmoid(f_ref[:, 0:_HID] + bz_ref[...])
        p = jnp.tanh(f_ref[:, 2 * _HID:3 * _HID] + bh_ref[...])
        out = z * p
        rid = lax.broadcasted_iota(jnp.int32, (blk, 1), 0) + i * blk
        o_ref[...] = jnp.where(rid == 0, 0.0, out)

    return pl.pallas_call(
        body,
        grid=(n // blk,),
        in_specs=[
            pl.BlockSpec((blk, 3 * _HID), lambda i: (i, 0)),
            pl.BlockSpec((1, _HID), lambda i: (0, 0)),
            pl.BlockSpec((1, _HID), lambda i: (0, 0)),
        ],
        out_specs=pl.BlockSpec((blk, _HID), lambda i: (i, 0)),
        out_shape=jax.ShapeDtypeStruct((n, _HID), jnp.float32),
    )(fmsg, bz, bh)


def _tc_gru_step(nei3, fmsg, wzs_t, ur_t, whs_t, bz, br, bh):
    n = fmsg.shape[0]
    blk = 1024

    def body(nei_ref, f_ref, wzs_ref, ur_ref, whs_ref, bz_ref, br_ref, bh_ref,
             o_ref):
        i = pl.program_id(0)
        nei = nei_ref[...]
        r2 = jnp.dot(nei, ur_ref[...], preferred_element_type=jnp.float32)
        n0 = nei[0:blk]
        n1 = nei[blk:2 * blk]
        n2 = nei[2 * blk:3 * blk]
        n3 = nei[3 * blk:4 * blk]
        sum_msg = n0 + n1 + n2 + n3
        rb = f_ref[:, _HID:2 * _HID] + br_ref[...]
        sg = jax.nn.sigmoid(rb + r2[0:blk]) * n0
        sg = sg + jax.nn.sigmoid(rb + r2[blk:2 * blk]) * n1
        sg = sg + jax.nn.sigmoid(rb + r2[2 * blk:3 * blk]) * n2
        sg = sg + jax.nn.sigmoid(rb + r2[3 * blk:4 * blk]) * n3
        z = jax.nn.sigmoid(
            f_ref[:, 0:_HID]
            + jnp.dot(sum_msg, wzs_ref[...], preferred_element_type=jnp.float32)
            + bz_ref[...]
        )
        pre = jnp.tanh(
            f_ref[:, 2 * _HID:3 * _HID]
            + jnp.dot(sg, whs_ref[...], preferred_element_type=jnp.float32)
            + bh_ref[...]
        )
        out = (1.0 - z) * sum_msg + z * pre
        rid = lax.broadcasted_iota(jnp.int32, (blk, 1), 0) + i * blk
        o_ref[...] = jnp.where(rid == 0, 0.0, out)

    return pl.pallas_call(
        body,
        grid=(n // blk,),
        in_specs=[
            pl.BlockSpec((4 * blk, _HID), lambda i: (i, 0)),
            pl.BlockSpec((blk, 3 * _HID), lambda i: (i, 0)),
            pl.BlockSpec((_HID, _HID), lambda i: (0, 0)),
            pl.BlockSpec((_HID, _HID), lambda i: (0, 0)),
            pl.BlockSpec((_HID, _HID), lambda i: (0, 0)),
            pl.BlockSpec((1, _HID), lambda i: (0, 0)),
            pl.BlockSpec((1, _HID), lambda i: (0, 0)),
            pl.BlockSpec((1, _HID), lambda i: (0, 0)),
        ],
        out_specs=pl.BlockSpec((blk, _HID), lambda i: (i, 0)),
        out_shape=jax.ShapeDtypeStruct((n, _HID), jnp.float32),
    )(nei3, fmsg, wzs_t, ur_t, whs_t, bz, br, bh)


def _tc_gru_last(nei3, fmsg, pout, wzs_t, ur_t, whs_t, or_t, bz, br, bh):
    n = fmsg.shape[0]
    blk = 1024

    def body(nei_ref, f_ref, p_ref, wzs_ref, ur_ref, whs_ref, or_ref,
             bz_ref, br_ref, bh_ref, o_ref, q_ref):
        i = pl.program_id(0)
        nei = nei_ref[...]
        r2 = jnp.dot(nei, ur_ref[...], preferred_element_type=jnp.float32)
        n0 = nei[0:blk]
        n1 = nei[blk:2 * blk]
        n2 = nei[2 * blk:3 * blk]
        n3 = nei[3 * blk:4 * blk]
        sum_msg = n0 + n1 + n2 + n3
        rb = f_ref[:, _HID:2 * _HID] + br_ref[...]
        sg = jax.nn.sigmoid(rb + r2[0:blk]) * n0
        sg = sg + jax.nn.sigmoid(rb + r2[blk:2 * blk]) * n1
        sg = sg + jax.nn.sigmoid(rb + r2[2 * blk:3 * blk]) * n2
        sg = sg + jax.nn.sigmoid(rb + r2[3 * blk:4 * blk]) * n3
        z = jax.nn.sigmoid(
            f_ref[:, 0:_HID]
            + jnp.dot(sum_msg, wzs_ref[...], preferred_element_type=jnp.float32)
            + bz_ref[...]
        )
        pre = jnp.tanh(
            f_ref[:, 2 * _HID:3 * _HID]
            + jnp.dot(sg, whs_ref[...], preferred_element_type=jnp.float32)
            + bh_ref[...]
        )
        out = (1.0 - z) * sum_msg + z * pre
        rid = lax.broadcasted_iota(jnp.int32, (blk, 1), 0) + i * blk
        out = jnp.where(rid == 0, 0.0, out)
        o_ref[...] = out
        q_ref[...] = p_ref[...] + jnp.dot(
            out, or_ref[...], preferred_element_type=jnp.float32)

    return pl.pallas_call(
        body,
        grid=(n // blk,),
        in_specs=[
            pl.BlockSpec((4 * blk, _HID), lambda i: (i, 0)),
            pl.BlockSpec((blk, 3 * _HID), lambda i: (i, 0)),
            pl.BlockSpec((blk, _HID), lambda i: (i, 0)),
            pl.BlockSpec((_HID, _HID), lambda i: (0, 0)),
            pl.BlockSpec((_HID, _HID), lambda i: (0, 0)),
            pl.BlockSpec((_HID, _HID), lambda i: (0, 0)),
            pl.BlockSpec((_HID, _HID), lambda i: (0, 0)),
            pl.BlockSpec((1, _HID), lambda i: (0, 0)),
            pl.BlockSpec((1, _HID), lambda i: (0, 0)),
            pl.BlockSpec((1, _HID), lambda i: (0, 0)),
        ],
        out_specs=[
            pl.BlockSpec((blk, _HID), lambda i: (i, 0)),
            pl.BlockSpec((blk, _HID), lambda i: (i, 0)),
        ],
        out_shape=[
            jax.ShapeDtypeStruct((n, _HID), jnp.float32),
            jax.ShapeDtypeStruct((n, _HID), jnp.float32),
        ],
    )(nei3, fmsg, pout, wzs_t, ur_t, whs_t, or_t, bz, br, bh)


def _tc_lstm(agg_tm, out_b, wf_t, wb_t, uf_t, ub_t, bf, bb):
    batch, seq, hh = 8, 512, 128

    def body(agg_ref, ob_ref, wf_ref, wb_ref, uf_ref, ub_ref,
             bf_ref, bb_ref, o_ref, xf_ref, xb_ref):
        nrow = batch * seq
        blk = 512
        for c in range(nrow // blk):
            h = jnp.maximum(agg_ref[pl.ds(c * blk, blk), :] + ob_ref[...], 0.0)
            xf_ref[pl.ds(c * blk, blk), :] = jnp.dot(
                h, wf_ref[...], preferred_element_type=jnp.float32) + bf_ref[...]
            xb_ref[pl.ds(c * blk, blk), :] = jnp.dot(
                h, wb_ref[...], preferred_element_type=jnp.float32) + bb_ref[...]

        uf_hi = uf_ref[...]
        ub_hi = ub_ref[...]

        def cell(g, c):
            i_ = jax.nn.sigmoid(g[:, 0:hh])
            f_ = jax.nn.sigmoid(g[:, hh:2 * hh])
            g_ = jnp.tanh(g[:, 2 * hh:3 * hh])
            o_ = jax.nn.sigmoid(g[:, 3 * hh:4 * hh])
            c = f_ * c + i_ * g_
            return o_ * jnp.tanh(c), c

        unroll = 16

        def step(i, carry):
            hf, cf, hb, cb = carry
            t0 = i * unroll
            xfb = xf_ref[pl.ds(t0 * batch, unroll * batch), :]
            xbb = xb_ref[pl.ds((seq - unroll - t0) * batch, unroll * batch), :]
            for k in range(unroll):
                kb = unroll - 1 - k
                gf = xfb[k * batch:(k + 1) * batch, :] + jnp.dot(
                    hf.astype(jnp.bfloat16), uf_hi,
                    preferred_element_type=jnp.float32)
                gb = xbb[kb * batch:(kb + 1) * batch, :] + jnp.dot(
                    hb.astype(jnp.bfloat16), ub_hi,
                    preferred_element_type=jnp.float32)
                hf, cf = cell(gf, cf)
                hb, cb = cell(gb, cb)
            return (hf, cf, hb, cb)

        z = jnp.zeros((batch, hh), jnp.float32)
        hf, cf, hb, cb = lax.fori_loop(0, seq // unroll, step, (z, z, z, z))
        o_ref[...] = jnp.concatenate([hf, hb], axis=1)

    return pl.pallas_call(
        body,
        in_specs=[
            pl.BlockSpec((batch * seq, _HID), lambda: (0, 0)),
            pl.BlockSpec((1, _HID), lambda: (0, 0)),
            pl.BlockSpec((_HID, 4 * hh), lambda: (0, 0)),
            pl.BlockSpec((_HID, 4 * hh), lambda: (0, 0)),
            pl.BlockSpec((hh, 4 * hh), lambda: (0, 0)),
            pl.BlockSpec((hh, 4 * hh), lambda: (0, 0)),
            pl.BlockSpec((1, 4 * hh), lambda: (0, 0)),
            pl.BlockSpec((1, 4 * hh), lambda: (0, 0)),
        ],
        out_specs=pl.BlockSpec((batch, 2 * hh), lambda: (0, 0)),
        out_shape=jax.ShapeDtypeStruct((batch, 2 * hh), jnp.float32),
        scratch_shapes=[
            pltpu.VMEM((seq * batch, 4 * hh), jnp.float32),
            pltpu.VMEM((seq * batch, 4 * hh), jnp.float32),
        ],
    )(agg_tm, out_b, wf_t, wb_t,
      uf_t.astype(jnp.bfloat16), ub_t.astype(jnp.bfloat16), bf, bb)


def kernel(nuc_emebedding, f_node_label, f_node_assignment, f_message,
           node_graph, message_graph, scope, diameter,
           W_z_w, W_z_b, W_r_w, U_r_w, U_r_b, W_h_w, W_h_b, out_w, out_b,
           lstm_Wih_f, lstm_Whh_f, lstm_bih_f, lstm_bhh_f,
           lstm_Wih_b, lstm_Whh_b, lstm_bih_b, lstm_bhh_b):
    hid = _HID

    nuc_pad = jnp.concatenate(
        [nuc_emebedding, jnp.zeros((1, hid), jnp.float32)], axis=0)
    fna = _sc_gather_sum(
        nuc_pad, f_node_assignment.astype(jnp.int32).reshape(-1), 8, 16)

    wcat = jnp.concatenate(
        [W_z_w[:, :_IN], W_r_w, W_h_w[:, :_IN], out_w[:, :_IN]], axis=0)
    wl_t = jnp.transpose(wcat[:, :_FD])
    we_t = jnp.transpose(wcat[:, _FD:])
    p768, pout = _tc_pmat(f_node_label, fna, wl_t, we_t)

    fmsg = _sc_gather_rows(p768, f_message.astype(jnp.int32), 32)

    bz = W_z_b.reshape(1, hid)
    br = U_r_b.reshape(1, hid)
    bh = W_h_b.reshape(1, hid)
    wzs_t = jnp.transpose(W_z_w[:, _IN:])
    ur_t = jnp.transpose(U_r_w)
    whs_t = jnp.transpose(W_h_w[:, _IN:])

    msgs = _tc_gru_first(fmsg, bz, bh)
    mg_flat = jnp.transpose(
        message_graph.astype(jnp.int32).reshape(8, 1024, 4), (0, 2, 1)).reshape(-1)
    for _ in range(3):
        nei = _sc_gather_rows(msgs, mg_flat, 64)
        msgs = _tc_gru_step(nei, fmsg, wzs_t, ur_t, whs_t, bz, br, bh)

    or_t = jnp.transpose(out_w[:, _IN:])
    nei = _sc_gather_rows(msgs, mg_flat, 64)
    msgs, q = _tc_gru_last(nei, fmsg, pout,
                           wzs_t, ur_t, whs_t, or_t, bz, br, bh)

    ng_tm = jnp.transpose(
        node_graph.astype(jnp.int32).reshape(8, 512, 4), (1, 0, 2)).reshape(-1)
    agg = _sc_gather_sum(q, ng_tm, 4, 32)

    tree = _tc_lstm(
        agg, out_b.reshape(1, hid),
        jnp.transpose(lstm_Wih_f), jnp.transpose(lstm_Wih_b),
        jnp.transpose(lstm_Whh_f), jnp.transpose(lstm_Whh_b),
        (lstm_bih_f + lstm_bhh_f).reshape(1, -1),
        (lstm_bih_b + lstm_bhh_b).reshape(1, -1))

    return (msgs, tree)


# speedup vs baseline: 1.0115x; 1.0003x over previous
import functools

import jax
import jax.numpy as jnp
from jax import lax
from jax.experimental import pallas as pl
from jax.experimental.pallas import tpu as pltpu
from jax.experimental.pallas import tpu_sc as plsc

_HID = 256
_FD = 4
_IN = _FD + _HID


def _sc_mesh_info():
    info = plsc.get_sparse_core_info()
    return info.num_cores, info.num_subcores


def _sc_gather_rows(table, idx, chunk):
    (n,) = idx.shape
    _, d = table.shape
    dt = table.dtype
    nc, ns = _sc_mesh_info()
    nw = nc * ns
    npw = n // nw
    nchunks = npw // chunk
    mesh = plsc.VectorSubcoreMesh(core_axis_name="c", subcore_axis_name="s")

    assert nchunks % 4 == 0

    @functools.partial(
        pl.kernel,
        mesh=mesh,
        out_type=jax.ShapeDtypeStruct((n, d), dt),
        scratch_types=[
            pltpu.VMEM((npw,), jnp.int32),
            pltpu.VMEM((chunk, d), dt),
            pltpu.VMEM((chunk, d), dt),
            pltpu.VMEM((chunk, d), dt),
            pltpu.VMEM((chunk, d), dt),
            pltpu.SemaphoreType.DMA,
            pltpu.SemaphoreType.DMA,
            pltpu.SemaphoreType.DMA,
            pltpu.SemaphoreType.DMA,
        ],
    )
    def k(table_hbm, idx_hbm, out_hbm, idx_v,
          rows0, rows1, rows2, rows3, g0, g1, g2, g3):
        wid = lax.axis_index("s") * nc + lax.axis_index("c")
        base = wid * npw
        pltpu.sync_copy(idx_hbm.at[pl.ds(base, npw)], idx_v)
        rows = (rows0, rows1, rows2, rows3)
        gs = (g0, g1, g2, g3)

        def start(c, b):
            pltpu.async_copy(
                table_hbm.at[idx_v.at[pl.ds(c * chunk, chunk)]], rows[b], gs[b])

        start(0, 0)
        start(1, 1)

        def quad(i, carry):
            qq = i * 4
            for b in range(4):
                c = qq + b

                @pl.when(c + 2 < nchunks)
                def _():
                    start(c + 2, (b + 2) % 4)

                pltpu.make_async_copy(
                    table_hbm.at[idx_v.at[pl.ds(c * chunk, chunk)]],
                    rows[b], gs[b]).wait()
                pltpu.sync_copy(rows[b], out_hbm.at[pl.ds(base + c * chunk, chunk)])
            return carry

        lax.fori_loop(0, nchunks // 4, quad, 0)

    return k(table, idx)


def _sc_gather_sum(table, idx, kper, cout):
    n = idx.shape[0] // kper
    _, d = table.shape
    nc, ns = _sc_mesh_info()
    nw = nc * ns
    npw = n // nw
    chunk = cout * kper
    nchunks = npw // cout
    mesh = plsc.VectorSubcoreMesh(core_axis_name="c", subcore_axis_name="s")

    @functools.partial(
        pl.kernel,
        mesh=mesh,
        out_type=jax.ShapeDtypeStruct((n, d), jnp.float32),
        scratch_types=[
            pltpu.VMEM((npw * kper,), jnp.int32),
            pltpu.VMEM((chunk, d), jnp.float32),
            pltpu.VMEM((chunk, d), jnp.float32),
            pltpu.VMEM((cout, d), jnp.float32),
            pltpu.SemaphoreType.DMA,
            pltpu.SemaphoreType.DMA,
        ],
    )
    def k(table_hbm, idx_hbm, out_hbm, idx_v, rows0, rows1, acc_v, g0, g1):
        wid = lax.axis_index("s") * nc + lax.axis_index("c")
        base = wid * npw
        pltpu.sync_copy(idx_hbm.at[pl.ds(base * kper, npw * kper)], idx_v)
        rows = (rows0, rows1)
        gs = (g0, g1)

        def start(c, b):
            pltpu.async_copy(
                table_hbm.at[idx_v.at[pl.ds(c * chunk, chunk)]], rows[b], gs[b])

        start(0, 0)

        def pair(i, carry):
            cc = i * 2
            for b in range(2):
                c = cc + b

                @pl.when(c + 1 < nchunks)
                def _():
                    start(c + 1, 1 - b)

                pltpu.make_async_copy(
                    table_hbm.at[idx_v.at[pl.ds(c * chunk, chunk)]],
                    rows[b], gs[b]).wait()

                def row(r, c2):
                    for lc in range(d // 16):
                        sl = pl.ds(lc * 16, 16)
                        a = rows[b][r * kper, sl]
                        for j in range(1, kper):
                            a = a + rows[b][r * kper + j, sl]
                        acc_v[r, sl] = a
                    return c2

                lax.fori_loop(0, cout, row, 0)
                pltpu.sync_copy(acc_v, out_hbm.at[pl.ds(base + c * cout, cout)])
            return carry

        lax.fori_loop(0, nchunks // 2, pair, 0)

    return k(table, idx)


def _tc_pmat(label, f_na, wl_t, we_t):
    n = f_na.shape[0]
    dout = we_t.shape[1]
    blk = 512
    ngrid = n // blk + 1

    def body(lab_ref, fna_ref, wl_ref, we_ref, o_ref, p_ref):
        i = pl.program_id(0)

        @pl.when(i < ngrid - 1)
        def _():
            p = (
                jnp.dot(lab_ref[...], wl_ref[...], preferred_element_type=jnp.float32)
                + jnp.dot(fna_ref[...], we_ref[...], preferred_element_type=jnp.float32)
            )
            o_ref[...] = p[:, :3 * _HID]
            p_ref[...] = p[:, 3 * _HID:]

        @pl.when(i == ngrid - 1)
        def _():
            o_ref[...] = jnp.zeros((blk, 3 * _HID), jnp.float32)
            p_ref[...] = jnp.zeros((blk, _HID), jnp.float32)

    return pl.pallas_call(
        body,
        grid=(ngrid,),
        in_specs=[
            pl.BlockSpec((blk, _FD), lambda i: (jnp.minimum(i, ngrid - 2), 0)),
            pl.BlockSpec((blk, _HID), lambda i: (jnp.minimum(i, ngrid - 2), 0)),
            pl.BlockSpec((_FD, dout), lambda i: (0, 0)),
            pl.BlockSpec((_HID, dout), lambda i: (0, 0)),
        ],
        out_specs=[
            pl.BlockSpec((blk, 3 * _HID), lambda i: (i, 0)),
            pl.BlockSpec((blk, _HID), lambda i: (i, 0)),
        ],
        out_shape=[
            jax.ShapeDtypeStruct((ngrid * blk, 3 * _HID), jnp.float32),
            jax.ShapeDtypeStruct((ngrid * blk, _HID), jnp.float32),
        ],
    )(label, f_na, wl_t, we_t)


def _tc_gru_first(fmsg, bz, bh):
    n = fmsg.shape[0]
    blk = 1024

    def body(f_ref, bz_ref, bh_ref, o_ref):
        i = pl.program_id(0)
        z = jax.nn.sigmoid(f_ref[:, 0:_HID] + bz_ref[...])
        p = jnp.tanh(f_ref[:, 2 * _HID:3 * _HID] + bh_ref[...])
        out = z * p
        rid = lax.broadcasted_iota(jnp.int32, (blk, 1), 0) + i * blk
        o_ref[...] = jnp.where(rid == 0, 0.0, out)

    return pl.pallas_call(
        body,
        grid=(n // blk,),
        in_specs=[
            pl.BlockSpec((blk, 3 * _HID), lambda i: (i, 0)),
            pl.BlockSpec((1, _HID), lambda i: (0, 0)),
            pl.BlockSpec((1, _HID), lambda i: (0, 0)),
        ],
        out_specs=pl.BlockSpec((blk, _HID), lambda i: (i, 0)),
        out_shape=jax.ShapeDtypeStruct((n, _HID), jnp.float32),
    )(fmsg, bz, bh)


def _tc_gru_step(nei3, fmsg, wzs_t, ur_t, whs_t, bz, br, bh):
    n = fmsg.shape[0]
    blk = 1024

    def body(nei_ref, f_ref, wzs_ref, ur_ref, whs_ref, bz_ref, br_ref, bh_ref,
             o_ref):
        i = pl.program_id(0)
        nei = nei_ref[...]
        r2 = jnp.dot(nei, ur_ref[...], preferred_element_type=jnp.float32)
        n0 = nei[0:blk]
        n1 = nei[blk:2 * blk]
        n2 = nei[2 * blk:3 * blk]
        n3 = nei[3 * blk:4 * blk]
        sum_msg = n0 + n1 + n2 + n3
        rb = f_ref[:, _HID:2 * _HID] + br_ref[...]
        sg = jax.nn.sigmoid(rb + r2[0:blk]) * n0
        sg = sg + jax.nn.sigmoid(rb + r2[blk:2 * blk]) * n1
        sg = sg + jax.nn.sigmoid(rb + r2[2 * blk:3 * blk]) * n2
        sg = sg + jax.nn.sigmoid(rb + r2[3 * blk:4 * blk]) * n3
        z = jax.nn.sigmoid(
            f_ref[:, 0:_HID]
            + jnp.dot(sum_msg, wzs_ref[...], preferred_element_type=jnp.float32)
            + bz_ref[...]
        )
        pre = jnp.tanh(
            f_ref[:, 2 * _HID:3 * _HID]
            + jnp.dot(sg, whs_ref[...], preferred_element_type=jnp.float32)
            + bh_ref[...]
        )
        out = (1.0 - z) * sum_msg + z * pre
        rid = lax.broadcasted_iota(jnp.int32, (blk, 1), 0) + i * blk
        o_ref[...] = jnp.where(rid == 0, 0.0, out)

    return pl.pallas_call(
        body,
        grid=(n // blk,),
        in_specs=[
            pl.BlockSpec((4 * blk, _HID), lambda i: (i, 0)),
            pl.BlockSpec((blk, 3 * _HID), lambda i: (i, 0)),
            pl.BlockSpec((_HID, _HID), lambda i: (0, 0)),
            pl.BlockSpec((_HID, _HID), lambda i: (0, 0)),
            pl.BlockSpec((_HID, _HID), lambda i: (0, 0)),
            pl.BlockSpec((1, _HID), lambda i: (0, 0)),
            pl.BlockSpec((1, _HID), lambda i: (0, 0)),
            pl.BlockSpec((1, _HID), lambda i: (0, 0)),
        ],
        out_specs=pl.BlockSpec((blk, _HID), lambda i: (i, 0)),
        out_shape=jax.ShapeDtypeStruct((n, _HID), jnp.float32),
    )(nei3, fmsg, wzs_t, ur_t, whs_t, bz, br, bh)


def _tc_gru_last(nei3, fmsg, pout, wzs_t, ur_t, whs_t, or_t, bz, br, bh):
    n = fmsg.shape[0]
    blk = 1024

    def body(nei_ref, f_ref, p_ref, wzs_ref, ur_ref, whs_ref, or_ref,
             bz_ref, br_ref, bh_ref, o_ref, q_ref):
        i = pl.program_id(0)
        nei = nei_ref[...]
        r2 = jnp.dot(nei, ur_ref[...], preferred_element_type=jnp.float32)
        n0 = nei[0:blk]
        n1 = nei[blk:2 * blk]
        n2 = nei[2 * blk:3 * blk]
        n3 = nei[3 * blk:4 * blk]
        sum_msg = n0 + n1 + n2 + n3
        rb = f_ref[:, _HID:2 * _HID] + br_ref[...]
        sg = jax.nn.sigmoid(rb + r2[0:blk]) * n0
        sg = sg + jax.nn.sigmoid(rb + r2[blk:2 * blk]) * n1
        sg = sg + jax.nn.sigmoid(rb + r2[2 * blk:3 * blk]) * n2
        sg = sg + jax.nn.sigmoid(rb + r2[3 * blk:4 * blk]) * n3
        z = jax.nn.sigmoid(
            f_ref[:, 0:_HID]
            + jnp.dot(sum_msg, wzs_ref[...], preferred_element_type=jnp.float32)
            + bz_ref[...]
        )
        pre = jnp.tanh(
            f_ref[:, 2 * _HID:3 * _HID]
            + jnp.dot(sg, whs_ref[...], preferred_element_type=jnp.float32)
            + bh_ref[...]
        )
        out = (1.0 - z) * sum_msg + z * pre
        rid = lax.broadcasted_iota(jnp.int32, (blk, 1), 0) + i * blk
        out = jnp.where(rid == 0, 0.0, out)
        o_ref[...] = out
        q_ref[...] = p_ref[...] + jnp.dot(
            out, or_ref[...], preferred_element_type=jnp.float32)

    return pl.pallas_call(
        body,
        grid=(n // blk,),
        in_specs=[
            pl.BlockSpec((4 * blk, _HID), lambda i: (i, 0)),
            pl.BlockSpec((blk, 3 * _HID), lambda i: (i, 0)),
            pl.BlockSpec((blk, _HID), lambda i: (i, 0)),
            pl.BlockSpec((_HID, _HID), lambda i: (0, 0)),
            pl.BlockSpec((_HID, _HID), lambda i: (0, 0)),
            pl.BlockSpec((_HID, _HID), lambda i: (0, 0)),
            pl.BlockSpec((_HID, _HID), lambda i: (0, 0)),
            pl.BlockSpec((1, _HID), lambda i: (0, 0)),
            pl.BlockSpec((1, _HID), lambda i: (0, 0)),
            pl.BlockSpec((1, _HID), lambda i: (0, 0)),
        ],
        out_specs=[
            pl.BlockSpec((blk, _HID), lambda i: (i, 0)),
            pl.BlockSpec((blk, _HID), lambda i: (i, 0)),
        ],
        out_shape=[
            jax.ShapeDtypeStruct((n, _HID), jnp.float32),
            jax.ShapeDtypeStruct((n, _HID), jnp.float32),
        ],
    )(nei3, fmsg, pout, wzs_t, ur_t, whs_t, or_t, bz, br, bh)


def _tc_lstm(agg_tm, out_b, wf_t, wb_t, uf_t, ub_t, bf, bb):
    batch, seq, hh = 8, 512, 128

    def body(agg_ref, ob_ref, wf_ref, wb_ref, uf_ref, ub_ref,
             bf_ref, bb_ref, o_ref, xf_ref, xb_ref):
        nrow = batch * seq
        blk = 512
        for c in range(nrow // blk):
            h = jnp.maximum(agg_ref[pl.ds(c * blk, blk), :] + ob_ref[...], 0.0)
            xf_ref[pl.ds(c * blk, blk), :] = jnp.dot(
                h, wf_ref[...], preferred_element_type=jnp.float32) + bf_ref[...]
            xb_ref[pl.ds(c * blk, blk), :] = jnp.dot(
                h, wb_ref[...], preferred_element_type=jnp.float32) + bb_ref[...]

        uf_hi = uf_ref[...]
        ub_hi = ub_ref[...]

        def cell(g, c):
            i_ = jax.nn.sigmoid(g[:, 0:hh])
            f_ = jax.nn.sigmoid(g[:, hh:2 * hh])
            g_ = jnp.tanh(g[:, 2 * hh:3 * hh])
            o_ = jax.nn.sigmoid(g[:, 3 * hh:4 * hh])
            c = f_ * c + i_ * g_
            return o_ * jnp.tanh(c), c

        unroll = 16

        def step(i, carry):
            hf, cf, hb, cb = carry
            t0 = i * unroll
            xfb = xf_ref[pl.ds(t0 * batch, unroll * batch), :]
            xbb = xb_ref[pl.ds((seq - unroll - t0) * batch, unroll * batch), :]
            for k in range(unroll):
                kb = unroll - 1 - k
                gf = xfb[k * batch:(k + 1) * batch, :] + jnp.dot(
                    hf.astype(jnp.bfloat16), uf_hi,
                    preferred_element_type=jnp.float32)
                gb = xbb[kb * batch:(kb + 1) * batch, :] + jnp.dot(
                    hb.astype(jnp.bfloat16), ub_hi,
                    preferred_element_type=jnp.float32)
                hf, cf = cell(gf, cf)
                hb, cb = cell(gb, cb)
            return (hf, cf, hb, cb)

        z = jnp.zeros((batch, hh), jnp.float32)
        hf, cf, hb, cb = lax.fori_loop(0, seq // unroll, step, (z, z, z, z))
        o_ref[...] = jnp.concatenate([hf, hb], axis=1)

    return pl.pallas_call(
        body,
        in_specs=[
            pl.BlockSpec((batch * seq, _HID), lambda: (0, 0)),
            pl.BlockSpec((1, _HID), lambda: (0, 0)),
            pl.BlockSpec((_HID, 4 * hh), lambda: (0, 0)),
            pl.BlockSpec((_HID, 4 * hh), lambda: (0, 0)),
            pl.BlockSpec((hh, 4 * hh), lambda: (0, 0)),
            pl.BlockSpec((hh, 4 * hh), lambda: (0, 0)),
            pl.BlockSpec((1, 4 * hh), lambda: (0, 0)),
            pl.BlockSpec((1, 4 * hh), lambda: (0, 0)),
        ],
        out_specs=pl.BlockSpec((batch, 2 * hh), lambda: (0, 0)),
        out_shape=jax.ShapeDtypeStruct((batch, 2 * hh), jnp.float32),
        scratch_shapes=[
            pltpu.VMEM((seq * batch, 4 * hh), jnp.float32),
            pltpu.VMEM((seq * batch, 4 * hh), jnp.float32),
        ],
    )(agg_tm, out_b, wf_t, wb_t,
      uf_t.astype(jnp.bfloat16), ub_t.astype(jnp.bfloat16), bf, bb)


def kernel(nuc_emebedding, f_node_label, f_node_assignment, f_message,
           node_graph, message_graph, scope, diameter,
           W_z_w, W_z_b, W_r_w, U_r_w, U_r_b, W_h_w, W_h_b, out_w, out_b,
           lstm_Wih_f, lstm_Whh_f, lstm_bih_f, lstm_bhh_f,
           lstm_Wih_b, lstm_Whh_b, lstm_bih_b, lstm_bhh_b):
    hid = _HID

    nuc_pad = jnp.concatenate(
        [nuc_emebedding, jnp.zeros((1, hid), jnp.float32)], axis=0)
    fna = _sc_gather_sum(
        nuc_pad, f_node_assignment.astype(jnp.int32).reshape(-1), 8, 16)

    wcat = jnp.concatenate(
        [W_z_w[:, :_IN], W_r_w, W_h_w[:, :_IN], out_w[:, :_IN]], axis=0)
    wl_t = jnp.transpose(wcat[:, :_FD])
    we_t = jnp.transpose(wcat[:, _FD:])
    p768, pout = _tc_pmat(f_node_label, fna, wl_t, we_t)

    fmsg = _sc_gather_rows(p768, f_message.astype(jnp.int32), 32)

    bz = W_z_b.reshape(1, hid)
    br = U_r_b.reshape(1, hid)
    bh = W_h_b.reshape(1, hid)
    wzs_t = jnp.transpose(W_z_w[:, _IN:])
    ur_t = jnp.transpose(U_r_w)
    whs_t = jnp.transpose(W_h_w[:, _IN:])

    msgs = _tc_gru_first(fmsg, bz, bh)
    mg_flat = jnp.transpose(
        message_graph.astype(jnp.int32).reshape(8, 1024, 4), (0, 2, 1)).reshape(-1)
    for _ in range(3):
        nei = _sc_gather_rows(msgs, mg_flat, 64)
        msgs = _tc_gru_step(nei, fmsg, wzs_t, ur_t, whs_t, bz, br, bh)

    or_t = jnp.transpose(out_w[:, _IN:])
    nei = _sc_gather_rows(msgs, mg_flat, 64)
    msgs, q = _tc_gru_last(nei, fmsg, pout,
                           wzs_t, ur_t, whs_t, or_t, bz, br, bh)

    ng_tm = jnp.transpose(
        node_graph.astype(jnp.int32).reshape(8, 512, 4), (1, 0, 2)).reshape(-1)
    agg = _sc_gather_sum(q, ng_tm, 4, 32)

    tree = _tc_lstm(
        agg, out_b.reshape(1, hid),
        jnp.transpose(lstm_Wih_f), jnp.transpose(lstm_Wih_b),
        jnp.transpose(lstm_Whh_f), jnp.transpose(lstm_Whh_b),
        (lstm_bih_f + lstm_bhh_f).reshape(1, -1),
        (lstm_bih_b + lstm_bhh_b).reshape(1, -1))

    return (msgs, tree)
